# baseline (speedup 1.0000x reference)
import functools

import jax
import jax.numpy as jnp
from jax import lax
from jax.experimental import pallas as pl
from jax.experimental.pallas import tpu as pltpu
from jax.experimental.pallas import tpu_sc as plsc

_THETA = 0.25
_L = 16
_NC = 2
_NS = 16
_NT = 5


@functools.partial(jax.jit, static_argnums=(2,))
def _segmin_edge_update(pair_flat, attr_flat, n_nodes):
    E = pair_flat.shape[0] // 2
    ept = E // (_NC * _NS)
    npad = ((n_nodes + _L * _NS - 1) // (_L * _NS)) * (_L * _NS)
    chunk = npad // _NS
    assert ept % (_L * _NT) == 0 and ept % 8 == 0

    mesh = plsc.VectorSubcoreMesh(core_axis_name="c", subcore_axis_name="s")
    cparams = pltpu.CompilerParams(needs_layout_passes=False)

    @functools.partial(
        pl.kernel,
        out_type=(
            jax.ShapeDtypeStruct((_NC * npad,), jnp.float32),
            jax.ShapeDtypeStruct((E,), jnp.float32),
        ),
        mesh=mesh,
        compiler_params=cparams,
        scratch_types=[
            pltpu.VMEM((ept,), jnp.int32),
            [pltpu.VMEM((4 * ept // 5,), jnp.float32) for _ in range(2)],
            pltpu.VMEM((ept,), jnp.float32),
            [pltpu.VMEM((npad,), jnp.float32) for _ in range(_NT)],
            pltpu.VMEM((npad,), jnp.float32),
            pltpu.VMEM_SHARED((_NS, npad), jnp.float32),
            pltpu.SemaphoreType.DMA,
            pltpu.SemaphoreType.DMA,
        ],
    )
    def k1(pair_hbm, attr_hbm, tab_hbm, acol_hbm, row_v, blks, a_v, tabs,
           tabm, sp_tab, sem1, sem2):
        cid = lax.axis_index("c")
        sid = lax.axis_index("s")
        wid = cid * _NS + sid
        base = wid * ept
        ch = ept // 5
        cw = 4 * ch

        cp_row = pltpu.async_copy(pair_hbm.at[pl.ds(base, ept)], row_v, sem1)
        cps = [pltpu.async_copy(attr_hbm.at[pl.ds(4 * base, cw)],
                                blks[0], sem2)]

        inf16 = jnp.full((_L,), jnp.inf, jnp.float32)

        def init_body(i, c):
            for t in tabs:
                t[pl.ds(i * _L, _L)] = inf16
            return c
        lax.fori_loop(0, npad // _L, init_body, 0)

        lane4 = lax.iota(jnp.int32, _L) * 4
        assert ch % (_L * _NT) == 0

        for c in range(5):
            if c + 1 < 5:
                cps.append(pltpu.async_copy(
                    attr_hbm.at[pl.ds(4 * base + (c + 1) * cw, cw)],
                    blks[(c + 1) % 2], sem2))
            cps[c].wait()
            blk_v = blks[c % 2]

            def deint_body(i, cc, blk_v=blk_v, c=c):
                b = i * _NT * _L
                gs = [plsc.load_gather(blk_v, [lane4 + (b + u * _L) * 4])
                      for u in range(_NT)]
                for u in range(_NT):
                    a_v[pl.ds(c * ch + b + u * _L, _L)] = gs[u]
                return cc
            lax.fori_loop(0, ch // (_L * _NT), deint_body, 0)
        cp_row.wait()

        cp_acol = pltpu.async_copy(a_v, acol_hbm.at[pl.ds(base, ept)], sem2)

        trips = ept // (_L * _NT)

        def sweep(_):
            def body(i, acc):
                b = i * _NT * _L
                idxs = [row_v[pl.ds(b + u * _L, _L)] for u in range(_NT)]
                avs = [a_v[pl.ds(b + u * _L, _L)] for u in range(_NT)]
                curs = [plsc.load_gather(tabs[u], [idxs[u]])
                        for u in range(_NT)]
                losts = [avs[u] < curs[u] for u in range(_NT)]
                for u in range(_NT):
                    plsc.store_scatter(tabs[u], [idxs[u]], avs[u],
                                       mask=losts[u])
                chks = [plsc.load_gather(tabs[u], [idxs[u]])
                        for u in range(_NT)]
                for u in range(_NT):
                    acc = acc | (avs[u] < chks[u])
                return acc
            return lax.fori_loop(0, trips, body, jnp.zeros((_L,), jnp.bool_))

        fail = sweep(0)
        lax.while_loop(lambda f: jnp.any(f), sweep, fail)

        def merge_body(j, c):
            jo = j * _L
            m0 = tabs[0][pl.ds(jo, _L)]
            for t in tabs[1:]:
                m0 = jnp.minimum(m0, t[pl.ds(jo, _L)])
            tabm[pl.ds(jo, _L)] = m0
            return c
        lax.fori_loop(0, npad // _L, merge_body, 0)

        pltpu.sync_copy(tabm, sp_tab.at[sid])
        plsc.subcore_barrier()

        cb = sid * chunk
        stage = tabs[0]
        descs = [pltpu.async_copy(sp_tab.at[r, pl.ds(cb, chunk)],
                                  stage.at[pl.ds(r * chunk, chunk)], sem1)
                 for r in range(_NS)]
        for d in descs:
            d.wait()

        res = tabs[1]

        def red_body(j, c):
            jo = j * _L
            m0 = stage[pl.ds(jo, _L)]
            for r in range(1, _NS):
                m0 = jnp.minimum(m0, stage[pl.ds(r * chunk + jo, _L)])
            res[pl.ds(cb + jo, _L)] = m0
            return c
        lax.fori_loop(0, chunk // _L, red_body, 0)

        pltpu.sync_copy(res.at[pl.ds(cb, chunk)],
                        tab_hbm.at[pl.ds(cid * npad + cb, chunk)])
        cp_acol.wait()

    @functools.partial(
        pl.kernel,
        out_type=jax.ShapeDtypeStruct((E,), jnp.float32),
        mesh=mesh,
        compiler_params=cparams,
        scratch_types=[
            pltpu.VMEM((_NC * npad,), jnp.float32),
            pltpu.VMEM((npad,), jnp.float32),
            pltpu.VMEM((ept,), jnp.int32),
            pltpu.VMEM((ept,), jnp.float32),
            pltpu.VMEM((ept,), jnp.float32),
            pltpu.SemaphoreType.DMA,
            pltpu.SemaphoreType.DMA,
        ],
    )
    def k2(tab_hbm, acol_hbm, pair_hbm, out_hbm, s2_v, tabm, row_v, a_v,
           w_v, sem1, sem2):
        cid = lax.axis_index("c")
        sid = lax.axis_index("s")
        wid = cid * _NS + sid
        base = wid * ept

        cp_tab = pltpu.async_copy(tab_hbm, s2_v, sem1)
        cp_row = pltpu.async_copy(pair_hbm.at[pl.ds(base, ept)], row_v, sem2)
        cp_a = pltpu.async_copy(acol_hbm.at[pl.ds(base, ept)], a_v, sem2)
        cp_tab.wait()

        def merge_body(j, c):
            jo = j * _L
            tabm[pl.ds(jo, _L)] = jnp.minimum(s2_v[pl.ds(jo, _L)],
                                              s2_v[pl.ds(npad + jo, _L)])
            return c
        lax.fori_loop(0, npad // _L, merge_body, 0)
        cp_row.wait()
        cp_a.wait()

        def p2_body(j, c):
            b = j * _NT * _L
            idxs = [row_v[pl.ds(b + u * _L, _L)] for u in range(_NT)]
            avs = [a_v[pl.ds(b + u * _L, _L)] for u in range(_NT)]
            ms = [plsc.load_gather(tabm, [idxs[u]]) for u in range(_NT)]
            for u in range(_NT):
                w_v[pl.ds(b + u * _L, _L)] = jnp.maximum(
                    avs[u] / ms[u] - _THETA, 0.0)
            return c
        lax.fori_loop(0, ept // (_L * _NT), p2_body, 0)

        pltpu.sync_copy(w_v, out_hbm.at[pl.ds(base, ept)])

    tab2, acol = k1(pair_flat, attr_flat)
    return k2(tab2, acol, pair_flat)


def kernel(vertex_attr, edgeij_pair, edge_attr):
    return _segmin_edge_update(edgeij_pair.reshape(-1),
                               edge_attr.reshape(-1),
                               vertex_attr.shape[0])


# speedup vs baseline: 3.3696x; 3.3696x over previous
import functools

import jax
import jax.numpy as jnp
from jax import lax
from jax.experimental import pallas as pl
from jax.experimental.pallas import tpu as pltpu
from jax.experimental.pallas import tpu_sc as plsc

_THETA = 0.25
_L = 16
_NC = 2
_NS = 16
_NT = 5


@functools.partial(jax.jit, static_argnums=(2,))
def _segmin_edge_update(row, a, n_nodes):
    E = row.shape[0]
    ept = E // (_NC * _NS)
    npad = ((n_nodes + _L * _NS - 1) // (_L * _NS)) * (_L * _NS)
    chunk = npad // _NS
    assert ept % (_L * _NT) == 0 and ept % 8 == 0

    mesh = plsc.VectorSubcoreMesh(core_axis_name="c", subcore_axis_name="s")
    cparams = pltpu.CompilerParams(needs_layout_passes=False)

    @functools.partial(
        pl.kernel,
        out_type=jax.ShapeDtypeStruct((_NC * npad,), jnp.float32),
        mesh=mesh,
        compiler_params=cparams,
        scratch_types=[
            pltpu.VMEM((ept,), jnp.int32),
            pltpu.VMEM((ept,), jnp.float32),
            [pltpu.VMEM((npad,), jnp.float32) for _ in range(_NT)],
            pltpu.VMEM((npad,), jnp.float32),
            pltpu.VMEM_SHARED((_NS, npad), jnp.float32),
            pltpu.SemaphoreType.DMA,
            pltpu.SemaphoreType.DMA,
        ],
    )
    def k1(row_hbm, a_hbm, tab_hbm, row_v, a_v, tabs, tabm, sp_tab,
           sem1, sem2):
        cid = lax.axis_index("c")
        sid = lax.axis_index("s")
        wid = cid * _NS + sid
        base = wid * ept

        cp_row = pltpu.async_copy(row_hbm.at[pl.ds(base, ept)], row_v, sem1)
        cp_a = pltpu.async_copy(a_hbm.at[pl.ds(base, ept)], a_v, sem2)

        inf16 = jnp.full((_L,), jnp.inf, jnp.float32)

        def init_body(i, c):
            for t in tabs:
                t[pl.ds(i * _L, _L)] = inf16
            return c
        lax.fori_loop(0, npad // _L, init_body, 0)
        cp_row.wait()
        cp_a.wait()

        trips = ept // (_L * _NT)

        def sweep(_):
            def body(i, acc):
                b = i * _NT * _L
                idxs = [row_v[pl.ds(b + u * _L, _L)] for u in range(_NT)]
                avs = [a_v[pl.ds(b + u * _L, _L)] for u in range(_NT)]
                curs = [plsc.load_gather(tabs[u], [idxs[u]])
                        for u in range(_NT)]
                losts = [avs[u] < curs[u] for u in range(_NT)]
                for u in range(_NT):
                    plsc.store_scatter(tabs[u], [idxs[u]], avs[u],
                                       mask=losts[u])
                chks = [plsc.load_gather(tabs[u], [idxs[u]])
                        for u in range(_NT)]
                for u in range(_NT):
                    acc = acc | (avs[u] < chks[u])
                return acc
            return lax.fori_loop(0, trips, body, jnp.zeros((_L,), jnp.bool_))

        fail = sweep(0)
        lax.while_loop(lambda f: jnp.any(f), sweep, fail)

        def merge_body(j, c):
            jo = j * _L
            m0 = tabs[0][pl.ds(jo, _L)]
            for t in tabs[1:]:
                m0 = jnp.minimum(m0, t[pl.ds(jo, _L)])
            tabm[pl.ds(jo, _L)] = m0
            return c
        lax.fori_loop(0, npad // _L, merge_body, 0)

        pltpu.sync_copy(tabm, sp_tab.at[sid])
        plsc.subcore_barrier()

        cb = sid * chunk
        stage = tabs[0]
        descs = [pltpu.async_copy(sp_tab.at[r, pl.ds(cb, chunk)],
                                  stage.at[pl.ds(r * chunk, chunk)], sem1)
                 for r in range(_NS)]
        for d in descs:
            d.wait()

        res = tabs[1]

        def red_body(j, c):
            jo = j * _L
            m0 = stage[pl.ds(jo, _L)]
            for r in range(1, _NS):
                m0 = jnp.minimum(m0, stage[pl.ds(r * chunk + jo, _L)])
            res[pl.ds(cb + jo, _L)] = m0
            return c
        lax.fori_loop(0, chunk // _L, red_body, 0)

        pltpu.sync_copy(res.at[pl.ds(cb, chunk)],
                        tab_hbm.at[pl.ds(cid * npad + cb, chunk)])

    @functools.partial(
        pl.kernel,
        out_type=jax.ShapeDtypeStruct((E,), jnp.float32),
        mesh=mesh,
        compiler_params=cparams,
        scratch_types=[
            pltpu.VMEM((_NC * npad,), jnp.float32),
            pltpu.VMEM((npad,), jnp.float32),
            pltpu.VMEM((ept,), jnp.int32),
            pltpu.VMEM((ept,), jnp.float32),
            pltpu.VMEM((ept,), jnp.float32),
            pltpu.SemaphoreType.DMA,
            pltpu.SemaphoreType.DMA,
        ],
    )
    def k2(tab_hbm, row_hbm, a_hbm, out_hbm, s2_v, tabm, row_v, a_v,
           w_v, sem1, sem2):
        cid = lax.axis_index("c")
        sid = lax.axis_index("s")
        wid = cid * _NS + sid
        base = wid * ept

        cp_tab = pltpu.async_copy(tab_hbm, s2_v, sem1)
        cp_row = pltpu.async_copy(row_hbm.at[pl.ds(base, ept)], row_v, sem2)
        cp_a = pltpu.async_copy(a_hbm.at[pl.ds(base, ept)], a_v, sem2)
        cp_tab.wait()

        def merge_body(j, c):
            jo = j * _L
            tabm[pl.ds(jo, _L)] = jnp.minimum(s2_v[pl.ds(jo, _L)],
                                              s2_v[pl.ds(npad + jo, _L)])
            return c
        lax.fori_loop(0, npad // _L, merge_body, 0)
        cp_row.wait()
        cp_a.wait()

        def p2_body(j, c):
            b = j * _NT * _L
            idxs = [row_v[pl.ds(b + u * _L, _L)] for u in range(_NT)]
            avs = [a_v[pl.ds(b + u * _L, _L)] for u in range(_NT)]
            ms = [plsc.load_gather(tabm, [idxs[u]]) for u in range(_NT)]
            for u in range(_NT):
                w_v[pl.ds(b + u * _L, _L)] = jnp.maximum(
                    avs[u] / ms[u] - _THETA, 0.0)
            return c
        lax.fori_loop(0, ept // (_L * _NT), p2_body, 0)

        pltpu.sync_copy(w_v, out_hbm.at[pl.ds(base, ept)])

    tab2 = k1(row, a)
    return k2(tab2, row, a)


def kernel(vertex_attr, edgeij_pair, edge_attr):
    return _segmin_edge_update(edgeij_pair[0], edge_attr[:, 0],
                               vertex_attr.shape[0])


# speedup vs baseline: 4.0416x; 1.1994x over previous
import functools

import jax
import jax.numpy as jnp
from jax import lax
from jax.experimental import pallas as pl
from jax.experimental.pallas import tpu as pltpu
from jax.experimental.pallas import tpu_sc as plsc

_THETA = 0.25
_L = 16
_NC = 2
_NS = 16
_NT = 5


def _row0_body(pair_ref, out_ref):
    out_ref[...] = pair_ref[0]


def _extract_row0(edgeij_pair):
    E = edgeij_pair.shape[1]
    return pl.pallas_call(
        _row0_body,
        out_shape=jax.ShapeDtypeStruct((E,), edgeij_pair.dtype),
    )(edgeij_pair)


@functools.partial(jax.jit, static_argnums=(2,))
def _segmin_edge_update(pair, a, n_nodes):
    row = _extract_row0(pair)
    E = row.shape[0]
    ept = E // (_NC * _NS)
    npad = ((n_nodes + _L * _NS - 1) // (_L * _NS)) * (_L * _NS)
    chunk = npad // _NS
    assert ept % (_L * _NT) == 0 and ept % 8 == 0

    mesh = plsc.VectorSubcoreMesh(core_axis_name="c", subcore_axis_name="s")
    cparams = pltpu.CompilerParams(needs_layout_passes=False)

    @functools.partial(
        pl.kernel,
        out_type=jax.ShapeDtypeStruct((_NC * npad,), jnp.float32),
        mesh=mesh,
        compiler_params=cparams,
        scratch_types=[
            pltpu.VMEM((ept,), jnp.int32),
            pltpu.VMEM((ept,), jnp.float32),
            [pltpu.VMEM((npad,), jnp.float32) for _ in range(_NT)],
            pltpu.VMEM((npad,), jnp.float32),
            pltpu.VMEM_SHARED((_NS, npad), jnp.float32),
            pltpu.SemaphoreType.DMA,
            pltpu.SemaphoreType.DMA,
        ],
    )
    def k1(row_hbm, a_hbm, tab_hbm, row_v, a_v, tabs, tabm, sp_tab,
           sem1, sem2):
        cid = lax.axis_index("c")
        sid = lax.axis_index("s")
        wid = cid * _NS + sid
        base = wid * ept

        cp_row = pltpu.async_copy(row_hbm.at[pl.ds(base, ept)], row_v, sem1)
        cp_a = pltpu.async_copy(a_hbm.at[pl.ds(base, ept)], a_v, sem2)

        inf16 = jnp.full((_L,), jnp.inf, jnp.float32)

        def init_body(i, c):
            for t in tabs:
                t[pl.ds(i * _L, _L)] = inf16
            return c
        lax.fori_loop(0, npad // _L, init_body, 0)
        cp_row.wait()
        cp_a.wait()

        trips = ept // (_L * _NT)

        def sweep(_):
            def body(i, acc):
                b = i * _NT * _L
                idxs = [row_v[pl.ds(b + u * _L, _L)] for u in range(_NT)]
                avs = [a_v[pl.ds(b + u * _L, _L)] for u in range(_NT)]
                curs = [plsc.load_gather(tabs[u], [idxs[u]])
                        for u in range(_NT)]
                losts = [avs[u] < curs[u] for u in range(_NT)]
                for u in range(_NT):
                    plsc.store_scatter(tabs[u], [idxs[u]], avs[u],
                                       mask=losts[u])
                chks = [plsc.load_gather(tabs[u], [idxs[u]])
                        for u in range(_NT)]
                for u in range(_NT):
                    acc = acc | (avs[u] < chks[u])
                return acc
            return lax.fori_loop(0, trips, body, jnp.zeros((_L,), jnp.bool_))

        fail = sweep(0)
        lax.while_loop(lambda f: jnp.any(f), sweep, fail)

        def merge_body(j, c):
            jo = j * _L
            m0 = tabs[0][pl.ds(jo, _L)]
            for t in tabs[1:]:
                m0 = jnp.minimum(m0, t[pl.ds(jo, _L)])
            tabm[pl.ds(jo, _L)] = m0
            return c
        lax.fori_loop(0, npad // _L, merge_body, 0)

        pltpu.sync_copy(tabm, sp_tab.at[sid])
        plsc.subcore_barrier()

        cb = sid * chunk
        stage = tabs[0]
        descs = [pltpu.async_copy(sp_tab.at[r, pl.ds(cb, chunk)],
                                  stage.at[pl.ds(r * chunk, chunk)], sem1)
                 for r in range(_NS)]
        for d in descs:
            d.wait()

        res = tabs[1]

        def red_body(j, c):
            jo = j * _L
            m0 = stage[pl.ds(jo, _L)]
            for r in range(1, _NS):
                m0 = jnp.minimum(m0, stage[pl.ds(r * chunk + jo, _L)])
            res[pl.ds(cb + jo, _L)] = m0
            return c
        lax.fori_loop(0, chunk // _L, red_body, 0)

        pltpu.sync_copy(res.at[pl.ds(cb, chunk)],
                        tab_hbm.at[pl.ds(cid * npad + cb, chunk)])

    @functools.partial(
        pl.kernel,
        out_type=jax.ShapeDtypeStruct((E,), jnp.float32),
        mesh=mesh,
        compiler_params=cparams,
        scratch_types=[
            pltpu.VMEM((_NC * npad,), jnp.float32),
            pltpu.VMEM((npad,), jnp.float32),
            pltpu.VMEM((ept,), jnp.int32),
            pltpu.VMEM((ept,), jnp.float32),
            pltpu.VMEM((ept,), jnp.float32),
            pltpu.SemaphoreType.DMA,
            pltpu.SemaphoreType.DMA,
        ],
    )
    def k2(tab_hbm, row_hbm, a_hbm, out_hbm, s2_v, tabm, row_v, a_v,
           w_v, sem1, sem2):
        cid = lax.axis_index("c")
        sid = lax.axis_index("s")
        wid = cid * _NS + sid
        base = wid * ept

        cp_tab = pltpu.async_copy(tab_hbm, s2_v, sem1)
        cp_row = pltpu.async_copy(row_hbm.at[pl.ds(base, ept)], row_v, sem2)
        cp_a = pltpu.async_copy(a_hbm.at[pl.ds(base, ept)], a_v, sem2)
        cp_tab.wait()

        def merge_body(j, c):
            jo = j * _L
            tabm[pl.ds(jo, _L)] = jnp.minimum(s2_v[pl.ds(jo, _L)],
                                              s2_v[pl.ds(npad + jo, _L)])
            return c
        lax.fori_loop(0, npad // _L, merge_body, 0)
        cp_row.wait()
        cp_a.wait()

        def p2_body(j, c):
            b = j * _NT * _L
            idxs = [row_v[pl.ds(b + u * _L, _L)] for u in range(_NT)]
            avs = [a_v[pl.ds(b + u * _L, _L)] for u in range(_NT)]
            ms = [plsc.load_gather(tabm, [idxs[u]]) for u in range(_NT)]
            for u in range(_NT):
                w_v[pl.ds(b + u * _L, _L)] = jnp.maximum(
                    avs[u] / ms[u] - _THETA, 0.0)
            return c
        lax.fori_loop(0, ept // (_L * _NT), p2_body, 0)

        pltpu.sync_copy(w_v, out_hbm.at[pl.ds(base, ept)])

    tab2 = k1(row, a)
    return k2(tab2, row, a)


def kernel(vertex_attr, edgeij_pair, edge_attr):
    return _segmin_edge_update(edgeij_pair, edge_attr[:, 0],
                               vertex_attr.shape[0])
